# Initial kernel scaffold; baseline (speedup 1.0000x reference)
import jax
import jax.numpy as jnp
from jax.experimental import pallas as pl


def kernel(x, edge_index, edge_attr, batch, time, theta, B):
    raise NotImplementedError("write your pallas kernel here")


# speedup vs baseline: 2929.9588x; 2929.9588x over previous
import functools

import jax
import jax.numpy as jnp
from jax import lax
from jax.experimental import pallas as pl
from jax.experimental.pallas import tpu as pltpu
from jax.experimental.pallas import tpu_sc as plsc

_RADIUS = 3.0
_NCELL = 16
_NSTEPS = 50
_NCORES = 2
_NSUB = 16
_NW = _NCORES * _NSUB
_LANES = 16
_U = 4


def _prep_tables(theta, ba, bb, time):
    c = theta.shape[0]

    def body(time_ref, theta_ref, ba_ref, bb_ref, a_ref, b_ref):
        dt = time_ref[0] / jnp.float32(_NSTEPS)
        dn = (((1,), (1,)), ((), ()))
        a = lax.dot_general(theta_ref[...], ba_ref[...], dn,
                            preferred_element_type=jnp.float32)
        b = lax.dot_general(theta_ref[...], bb_ref[...], dn,
                            preferred_element_type=jnp.float32)
        a_ref[...] = jnp.float32(1.0) + dt * a
        b_ref[...] = (jnp.float32(16.0) * dt) * b

    return pl.pallas_call(
        body,
        in_specs=[
            pl.BlockSpec(memory_space=pltpu.SMEM),
            pl.BlockSpec(memory_space=pltpu.VMEM),
            pl.BlockSpec(memory_space=pltpu.VMEM),
            pl.BlockSpec(memory_space=pltpu.VMEM),
        ],
        out_specs=[
            pl.BlockSpec(memory_space=pltpu.VMEM),
            pl.BlockSpec(memory_space=pltpu.VMEM),
        ],
        out_shape=[
            jax.ShapeDtypeStruct((c, _NCELL), jnp.float32),
            jax.ShapeDtypeStruct((c, _NCELL), jnp.float32),
        ],
    )(time, theta, ba, bb)


def _sc_transform(xflat, a1flat, b16flat, nchan):
    n_elem = xflat.shape[0]
    chunk = n_elem // _NW
    assert chunk * _NW == n_elem and chunk % (_LANES * _U) == 0
    ngroups = chunk // (_LANES * _U)
    chmask = nchan - 1

    mesh = plsc.VectorSubcoreMesh(
        core_axis_name="c", subcore_axis_name="s",
        num_cores=_NCORES, num_subcores=_NSUB)

    @functools.partial(
        pl.kernel,
        mesh=mesh,
        compiler_params=pltpu.CompilerParams(needs_layout_passes=False),
        out_type=jax.ShapeDtypeStruct((n_elem,), jnp.float32),
        scratch_types=[
            pltpu.VMEM((chunk,), jnp.float32),
            pltpu.VMEM((chunk,), jnp.float32),
            pltpu.VMEM((nchan * _NCELL,), jnp.float32),
            pltpu.VMEM((nchan * _NCELL,), jnp.float32),
        ],
    )
    def run(x_hbm, a_hbm, b_hbm, out_hbm, xin, xout, atab, btab):
        wid = lax.axis_index("s") * _NCORES + lax.axis_index("c")
        base = wid * chunk
        pltpu.sync_copy(x_hbm.at[pl.ds(base, chunk)], xin)
        pltpu.sync_copy(a_hbm, atab)
        pltpu.sync_copy(b_hbm, btab)

        iota16 = lax.iota(jnp.int32, _LANES) * _NCELL

        def group(g, _):
            offs = g * (_LANES * _U)
            xv, msk, z, ib = [], [], [], []
            for u in range(_U):
                o = offs + u * _LANES
                x_u = xin[pl.ds(o, _LANES)]
                xs = (x_u + _RADIUS) / (2.0 * _RADIUS)
                xv.append(x_u)
                msk.append(jnp.logical_or(xs >= 1.0, xs <= 0.0))
                z.append(xs * jnp.float32(_NCELL))
                chbase = (base + o) & chmask
                ib.append(iota16 + chbase * _NCELL)

            def step(i, zs):
                out = []
                for u in range(_U):
                    zu = zs[u]
                    cell = jnp.minimum(
                        jnp.maximum(zu, jnp.float32(0.0)),
                        jnp.float32(_NCELL - 1)).astype(jnp.int32)
                    idx = cell + ib[u]
                    ac = plsc.load_gather(atab, [idx])
                    bc = plsc.load_gather(btab, [idx])
                    out.append(ac * zu + bc)
                return tuple(out)

            zf = lax.fori_loop(0, _NSTEPS, step, tuple(z))

            scale = jnp.float32(2.0 * _RADIUS / _NCELL)
            for u in range(_U):
                res = zf[u] * scale - jnp.float32(_RADIUS)
                res = jnp.where(msk[u], xv[u], res)
                xout[pl.ds(offs + u * _LANES, _LANES)] = res
            return 0

        lax.fori_loop(0, ngroups, group, 0)
        pltpu.sync_copy(xout, out_hbm.at[pl.ds(base, chunk)])

    return run(xflat, a1flat, b16flat)


def kernel(x, edge_index, edge_attr, batch, time, theta, B):
    n, nchan = x.shape
    ba = B[0::2, :]
    bb = B[1::2, :]
    a1, b16 = _prep_tables(theta, ba, bb, time)
    yflat = _sc_transform(x.reshape(-1), a1.reshape(-1), b16.reshape(-1), nchan)
    return (yflat.reshape(n, nchan), theta)


# speedup vs baseline: 4535.1439x; 1.5479x over previous
import functools

import jax
import jax.numpy as jnp
from jax import lax
from jax.experimental import pallas as pl
from jax.experimental.pallas import tpu as pltpu
from jax.experimental.pallas import tpu_sc as plsc

_RADIUS = 3.0
_NCELL = 16
_NSTEPS = 50
_NCORES = 2
_NSUB = 16
_NW = _NCORES * _NSUB
_LANES = 16
_U = 10
_STEP_UNROLL = 5


def _prep_tables(theta, ba, bb, time):
    c = theta.shape[0]

    def body(time_ref, theta_ref, ba_ref, bb_ref, a_ref, b_ref):
        dt = time_ref[0] / jnp.float32(_NSTEPS)
        dn = (((1,), (1,)), ((), ()))
        a = lax.dot_general(theta_ref[...], ba_ref[...], dn,
                            preferred_element_type=jnp.float32)
        b = lax.dot_general(theta_ref[...], bb_ref[...], dn,
                            preferred_element_type=jnp.float32)
        a_ref[...] = jnp.float32(1.0) + dt * a
        b_ref[...] = (jnp.float32(16.0) * dt) * b

    return pl.pallas_call(
        body,
        in_specs=[
            pl.BlockSpec(memory_space=pltpu.SMEM),
            pl.BlockSpec(memory_space=pltpu.VMEM),
            pl.BlockSpec(memory_space=pltpu.VMEM),
            pl.BlockSpec(memory_space=pltpu.VMEM),
        ],
        out_specs=[
            pl.BlockSpec(memory_space=pltpu.VMEM),
            pl.BlockSpec(memory_space=pltpu.VMEM),
        ],
        out_shape=[
            jax.ShapeDtypeStruct((c, _NCELL), jnp.float32),
            jax.ShapeDtypeStruct((c, _NCELL), jnp.float32),
        ],
    )(time, theta, ba, bb)


def _sc_transform(xflat, a1flat, b16flat, nchan):
    n_elem = xflat.shape[0]
    chunk = n_elem // _NW
    assert chunk * _NW == n_elem and chunk % (_LANES * _U) == 0
    ngroups = chunk // (_LANES * _U)
    chmask = nchan - 1

    mesh = plsc.VectorSubcoreMesh(
        core_axis_name="c", subcore_axis_name="s",
        num_cores=_NCORES, num_subcores=_NSUB)

    @functools.partial(
        pl.kernel,
        mesh=mesh,
        compiler_params=pltpu.CompilerParams(needs_layout_passes=False),
        out_type=jax.ShapeDtypeStruct((n_elem,), jnp.float32),
        scratch_types=[
            pltpu.VMEM((chunk,), jnp.float32),
            pltpu.VMEM((chunk,), jnp.float32),
            pltpu.VMEM((nchan * _NCELL,), jnp.float32),
            pltpu.VMEM((nchan * _NCELL,), jnp.float32),
        ],
    )
    def run(x_hbm, a_hbm, b_hbm, out_hbm, xin, xout, atab, btab):
        wid = lax.axis_index("s") * _NCORES + lax.axis_index("c")
        base = wid * chunk
        pltpu.sync_copy(x_hbm.at[pl.ds(base, chunk)], xin)
        pltpu.sync_copy(a_hbm, atab)
        pltpu.sync_copy(b_hbm, btab)

        iota16 = lax.iota(jnp.int32, _LANES) * _NCELL

        def group(g, _):
            offs = g * (_LANES * _U)
            xv, msk, z, ib = [], [], [], []
            for u in range(_U):
                o = offs + u * _LANES
                x_u = xin[pl.ds(o, _LANES)]
                xs = (x_u + _RADIUS) / (2.0 * _RADIUS)
                xv.append(x_u)
                msk.append(jnp.logical_or(xs >= 1.0, xs <= 0.0))
                z.append(xs * jnp.float32(_NCELL))
                chbase = (base + o) & chmask
                ib.append(iota16 + chbase * _NCELL)

            def step(i, zs):
                zs = list(zs)
                for _ in range(_STEP_UNROLL):
                    out = []
                    for u in range(_U):
                        zu = zs[u]
                        cell = jnp.minimum(
                            jnp.maximum(zu, jnp.float32(0.0)),
                            jnp.float32(_NCELL - 1)).astype(jnp.int32)
                        idx = cell + ib[u]
                        ac = plsc.load_gather(atab, [idx])
                        bc = plsc.load_gather(btab, [idx])
                        out.append(ac * zu + bc)
                    zs = out
                return tuple(zs)

            zf = lax.fori_loop(0, _NSTEPS // _STEP_UNROLL, step, tuple(z))

            scale = jnp.float32(2.0 * _RADIUS / _NCELL)
            for u in range(_U):
                res = zf[u] * scale - jnp.float32(_RADIUS)
                res = jnp.where(msk[u], xv[u], res)
                xout[pl.ds(offs + u * _LANES, _LANES)] = res
            return 0

        lax.fori_loop(0, ngroups, group, 0)
        pltpu.sync_copy(xout, out_hbm.at[pl.ds(base, chunk)])

    return run(xflat, a1flat, b16flat)


def kernel(x, edge_index, edge_attr, batch, time, theta, B):
    n, nchan = x.shape
    ba = B[0::2, :]
    bb = B[1::2, :]
    a1, b16 = _prep_tables(theta, ba, bb, time)
    yflat = _sc_transform(x.reshape(-1), a1.reshape(-1), b16.reshape(-1), nchan)
    return (yflat.reshape(n, nchan), theta)


# speedup vs baseline: 4556.8242x; 1.0048x over previous
import functools

import jax
import jax.numpy as jnp
from jax import lax
from jax.experimental import pallas as pl
from jax.experimental.pallas import tpu as pltpu
from jax.experimental.pallas import tpu_sc as plsc

_RADIUS = 3.0
_NCELL = 16
_NSTEPS = 50
_NCORES = 2
_NSUB = 16
_NW = _NCORES * _NSUB
_LANES = 16
_U = 16
_STEP_UNROLL = 5
_BLK = _U * _LANES


def _prep_tables(theta, ba, bb, time):
    c = theta.shape[0]

    def body(time_ref, theta_ref, ba_ref, bb_ref, a_ref, b_ref):
        dt = time_ref[0] / jnp.float32(_NSTEPS)
        dn = (((1,), (1,)), ((), ()))
        a = lax.dot_general(theta_ref[...], ba_ref[...], dn,
                            preferred_element_type=jnp.float32)
        b = lax.dot_general(theta_ref[...], bb_ref[...], dn,
                            preferred_element_type=jnp.float32)
        a_ref[...] = jnp.float32(1.0) + dt * a
        b_ref[...] = (jnp.float32(16.0) * dt) * b

    return pl.pallas_call(
        body,
        in_specs=[
            pl.BlockSpec(memory_space=pltpu.SMEM),
            pl.BlockSpec(memory_space=pltpu.VMEM),
            pl.BlockSpec(memory_space=pltpu.VMEM),
            pl.BlockSpec(memory_space=pltpu.VMEM),
        ],
        out_specs=[
            pl.BlockSpec(memory_space=pltpu.VMEM),
            pl.BlockSpec(memory_space=pltpu.VMEM),
        ],
        out_shape=[
            jax.ShapeDtypeStruct((c, _NCELL), jnp.float32),
            jax.ShapeDtypeStruct((c, _NCELL), jnp.float32),
        ],
    )(time, theta, ba, bb)


def _sc_transform(xflat, a1flat, b16flat, nchan):
    n_elem = xflat.shape[0]
    nblk = n_elem // _BLK
    assert nblk * _BLK == n_elem
    base_blocks = nblk // _NW
    extra = nblk - base_blocks * _NW
    buf_words = (base_blocks + (1 if extra else 0)) * _BLK

    mesh = plsc.VectorSubcoreMesh(
        core_axis_name="c", subcore_axis_name="s",
        num_cores=_NCORES, num_subcores=_NSUB)

    @functools.partial(
        pl.kernel,
        mesh=mesh,
        compiler_params=pltpu.CompilerParams(needs_layout_passes=False),
        out_type=jax.ShapeDtypeStruct((n_elem,), jnp.float32),
        scratch_types=[
            pltpu.VMEM((buf_words,), jnp.float32),
            pltpu.VMEM((buf_words,), jnp.float32),
            pltpu.VMEM((nchan * _NCELL,), jnp.float32),
            pltpu.VMEM((nchan * _NCELL,), jnp.float32),
        ],
    )
    def run(x_hbm, a_hbm, b_hbm, out_hbm, xin, xout, atab, btab):
        wid = lax.axis_index("s") * _NCORES + lax.axis_index("c")
        is_big = wid < extra
        myblocks = base_blocks + jnp.where(is_big, 1, 0)
        start = wid * base_blocks + jnp.minimum(wid, extra)
        base = start * _BLK

        @pl.when(is_big)
        def _():
            pltpu.sync_copy(x_hbm.at[pl.ds(base, buf_words)], xin)

        @pl.when(jnp.logical_not(is_big))
        def _():
            pltpu.sync_copy(
                x_hbm.at[pl.ds(base, base_blocks * _BLK)],
                xin.at[pl.ds(0, base_blocks * _BLK)])

        pltpu.sync_copy(a_hbm, atab)
        pltpu.sync_copy(b_hbm, btab)

        iota16 = lax.iota(jnp.int32, _LANES) * _NCELL
        ib8 = [iota16 + (16 * j % nchan) * _NCELL for j in range(8)]

        def group(g, _):
            offs = g * _BLK
            z = []
            for u in range(_U):
                xs = (xin[pl.ds(offs + u * _LANES, _LANES)]
                      + _RADIUS) / (2.0 * _RADIUS)
                z.append(xs * jnp.float32(_NCELL))

            def step(i, zs):
                zs = list(zs)
                for _ in range(_STEP_UNROLL):
                    out = []
                    for u in range(_U):
                        zu = zs[u]
                        cell = jnp.minimum(
                            jnp.maximum(zu, jnp.float32(0.0)),
                            jnp.float32(_NCELL - 1)).astype(jnp.int32)
                        idx = cell + ib8[u % 8]
                        ac = plsc.load_gather(atab, [idx])
                        bc = plsc.load_gather(btab, [idx])
                        out.append(ac * zu + bc)
                    zs = out
                return tuple(zs)

            zf = lax.fori_loop(0, _NSTEPS // _STEP_UNROLL, step, tuple(z))

            scale = jnp.float32(2.0 * _RADIUS / _NCELL)
            for u in range(_U):
                xv = xin[pl.ds(offs + u * _LANES, _LANES)]
                xs = (xv + _RADIUS) / (2.0 * _RADIUS)
                msk = jnp.logical_or(xs >= 1.0, xs <= 0.0)
                res = zf[u] * scale - jnp.float32(_RADIUS)
                xout[pl.ds(offs + u * _LANES, _LANES)] = jnp.where(msk, xv, res)
            return 0

        lax.fori_loop(0, myblocks, group, 0)

        @pl.when(is_big)
        def _():
            pltpu.sync_copy(xout, out_hbm.at[pl.ds(base, buf_words)])

        @pl.when(jnp.logical_not(is_big))
        def _():
            pltpu.sync_copy(
                xout.at[pl.ds(0, base_blocks * _BLK)],
                out_hbm.at[pl.ds(base, base_blocks * _BLK)])

    return run(xflat, a1flat, b16flat)


def kernel(x, edge_index, edge_attr, batch, time, theta, B):
    n, nchan = x.shape
    ba = B[0::2, :]
    bb = B[1::2, :]
    a1, b16 = _prep_tables(theta, ba, bb, time)
    yflat = _sc_transform(x.reshape(-1), a1.reshape(-1), b16.reshape(-1), nchan)
    return (yflat.reshape(n, nchan), theta)


# speedup vs baseline: 5813.0967x; 1.2757x over previous
import functools

import jax
import jax.numpy as jnp
from jax import lax
from jax.experimental import pallas as pl
from jax.experimental.pallas import tpu as pltpu
from jax.experimental.pallas import tpu_sc as plsc

_RADIUS = 3.0
_NCELL = 16
_NSTEPS = 50
_NCORES = 2
_NSUB = 16
_NW = _NCORES * _NSUB
_LANES = 16
_U = 16
_STEP_UNROLL = 5
_BLK = _U * _LANES


def _prep_tables(theta, ba, bb, time):
    c = theta.shape[0]

    def body(time_ref, theta_ref, ba_ref, bb_ref, a_ref, b_ref):
        dt = time_ref[0] / jnp.float32(_NSTEPS)
        dn = (((1,), (1,)), ((), ()))
        a = lax.dot_general(ba_ref[...], theta_ref[...], dn,
                            preferred_element_type=jnp.float32)
        b = lax.dot_general(bb_ref[...], theta_ref[...], dn,
                            preferred_element_type=jnp.float32)
        a_ref[...] = jnp.float32(1.0) + dt * a
        b_ref[...] = (jnp.float32(16.0) * dt) * b

    return pl.pallas_call(
        body,
        in_specs=[
            pl.BlockSpec(memory_space=pltpu.SMEM),
            pl.BlockSpec(memory_space=pltpu.VMEM),
            pl.BlockSpec(memory_space=pltpu.VMEM),
            pl.BlockSpec(memory_space=pltpu.VMEM),
        ],
        out_specs=[
            pl.BlockSpec(memory_space=pltpu.VMEM),
            pl.BlockSpec(memory_space=pltpu.VMEM),
        ],
        out_shape=[
            jax.ShapeDtypeStruct((_NCELL, c), jnp.float32),
            jax.ShapeDtypeStruct((_NCELL, c), jnp.float32),
        ],
    )(time, theta, ba, bb)


def _sc_transform(xflat, a1flat, b16flat, nchan):
    n_elem = xflat.shape[0]
    nblk = n_elem // _BLK
    assert nblk * _BLK == n_elem
    base_blocks = nblk // _NW
    extra = nblk - base_blocks * _NW
    buf_words = (base_blocks + (1 if extra else 0)) * _BLK

    mesh = plsc.VectorSubcoreMesh(
        core_axis_name="c", subcore_axis_name="s",
        num_cores=_NCORES, num_subcores=_NSUB)

    @functools.partial(
        pl.kernel,
        mesh=mesh,
        compiler_params=pltpu.CompilerParams(needs_layout_passes=False),
        out_type=jax.ShapeDtypeStruct((n_elem,), jnp.float32),
        scratch_types=[
            pltpu.VMEM((buf_words,), jnp.float32),
            pltpu.VMEM((buf_words,), jnp.float32),
            pltpu.VMEM((nchan * _NCELL,), jnp.float32),
            pltpu.VMEM((nchan * _NCELL,), jnp.float32),
        ],
    )
    def run(x_hbm, a_hbm, b_hbm, out_hbm, xin, xout, atab, btab):
        wid = lax.axis_index("s") * _NCORES + lax.axis_index("c")
        is_big = wid < extra
        myblocks = base_blocks + jnp.where(is_big, 1, 0)
        start = wid * base_blocks + jnp.minimum(wid, extra)
        base = start * _BLK

        @pl.when(is_big)
        def _():
            pltpu.sync_copy(x_hbm.at[pl.ds(base, buf_words)], xin)

        @pl.when(jnp.logical_not(is_big))
        def _():
            pltpu.sync_copy(
                x_hbm.at[pl.ds(base, base_blocks * _BLK)],
                xin.at[pl.ds(0, base_blocks * _BLK)])

        pltpu.sync_copy(a_hbm, atab)
        pltpu.sync_copy(b_hbm, btab)

        iota = lax.iota(jnp.int32, _LANES)
        ib8 = [iota + (16 * j % nchan) for j in range(8)]

        def group(g, _):
            offs = g * _BLK
            z = []
            for u in range(_U):
                xs = (xin[pl.ds(offs + u * _LANES, _LANES)]
                      + _RADIUS) / (2.0 * _RADIUS)
                z.append(xs * jnp.float32(_NCELL))

            def step(i, zs):
                zs = list(zs)
                for _ in range(_STEP_UNROLL):
                    out = []
                    for u in range(_U):
                        zu = zs[u]
                        cell = jnp.minimum(
                            jnp.maximum(zu, jnp.float32(0.0)),
                            jnp.float32(_NCELL - 1)).astype(jnp.int32)
                        idx = cell * nchan + ib8[u % 8]
                        ac = plsc.load_gather(atab, [idx])
                        bc = plsc.load_gather(btab, [idx])
                        out.append(ac * zu + bc)
                    zs = out
                return tuple(zs)

            zf = lax.fori_loop(0, _NSTEPS // _STEP_UNROLL, step, tuple(z))

            scale = jnp.float32(2.0 * _RADIUS / _NCELL)
            for u in range(_U):
                xv = xin[pl.ds(offs + u * _LANES, _LANES)]
                xs = (xv + _RADIUS) / (2.0 * _RADIUS)
                msk = jnp.logical_or(xs >= 1.0, xs <= 0.0)
                res = zf[u] * scale - jnp.float32(_RADIUS)
                xout[pl.ds(offs + u * _LANES, _LANES)] = jnp.where(msk, xv, res)
            return 0

        lax.fori_loop(0, myblocks, group, 0)

        @pl.when(is_big)
        def _():
            pltpu.sync_copy(xout, out_hbm.at[pl.ds(base, buf_words)])

        @pl.when(jnp.logical_not(is_big))
        def _():
            pltpu.sync_copy(
                xout.at[pl.ds(0, base_blocks * _BLK)],
                out_hbm.at[pl.ds(base, base_blocks * _BLK)])

    return run(xflat, a1flat, b16flat)


def kernel(x, edge_index, edge_attr, batch, time, theta, B):
    n, nchan = x.shape
    ba = B[0::2, :]
    bb = B[1::2, :]
    a1, b16 = _prep_tables(theta, ba, bb, time)
    yflat = _sc_transform(x.reshape(-1), a1.reshape(-1), b16.reshape(-1), nchan)
    return (yflat.reshape(n, nchan), theta)
